# baseline (speedup 1.0000x reference)
import functools

import jax
import jax.numpy as jnp
from jax import lax
from jax.experimental import pallas as pl
from jax.experimental.pallas import tpu as pltpu
from jax.experimental.pallas import tpu_sc as plsc

N = 10000
E = 320000
F = 128
H = 128
C = 32
G = 64

NC = 2
NS = 16
NW = NC * NS
EPW = E // NW
CH = 25
NCH = EPW // CH
SCH = 40
NSB = NCH // SCH
KB_ = 5
NP = 10240
RPT = NP // NS

_f32 = jnp.float32


@functools.cache
def _make_sc_agg(with_counts: bool):
  mesh = plsc.VectorSubcoreMesh(core_axis_name="c", subcore_axis_name="s")
  out_type = [jax.ShapeDtypeStruct((NC, NP, F), _f32)]
  scratch = [
      pltpu.VMEM((SCH, CH), jnp.int32),
      pltpu.VMEM((SCH, CH), jnp.int32),
  ] + [
      pltpu.VMEM((CH, F), _f32) for _ in range(2 * KB_)
  ] + [
      pltpu.VMEM_SHARED((NP, F), _f32),
      pltpu.SemaphoreType.DMA,
      pltpu.SemaphoreType.DMA,
      pltpu.SemaphoreType.DMA,
      pltpu.SemaphoreType.DMA,
  ]
  if with_counts:
    out_type.append(jax.ShapeDtypeStruct((NC, NP, 16), _f32))
    scratch += [
        pltpu.VMEM((CH, 16), _f32),
        pltpu.VMEM((CH, 16), _f32),
        pltpu.VMEM_SHARED((NP, 16), _f32),
        pltpu.SemaphoreType.DMA,
    ]

  def body(src_hbm, dst_hbm, h_hbm, zrows_hbm, zcnt_hbm, ones_hbm, *refs):
    if with_counts:
      (acc_out, cnt_out, src_v, dst_v, *rest) = refs
      rows = rest[:2 * KB_]
      (acc_sh, sem_ga, sem_gb, sem_sa, sem_sb,
       ones_v, zc_v, cnt_sh, sem_c) = rest[2 * KB_:]
    else:
      (acc_out, src_v, dst_v, *rest) = refs
      rows = rest[:2 * KB_]
      acc_sh, sem_ga, sem_gb, sem_sa, sem_sb = rest[2 * KB_:]
    c = lax.axis_index("c")
    s = lax.axis_index("s")
    wid = c * NS + s
    base = s * RPT

    pltpu.sync_copy(zrows_hbm, acc_sh.at[pl.ds(base, RPT)])
    if with_counts:
      pltpu.sync_copy(zcnt_hbm, cnt_sh.at[pl.ds(base, RPT)])
      pltpu.sync_copy(ones_hbm, ones_v)
    plsc.subcore_barrier()

    @pl.loop(0, NSB)
    def superblock(sb):
      ix0 = pltpu.async_copy(src_hbm.at[wid, sb], src_v, sem_ga)
      ix1 = pltpu.async_copy(dst_hbm.at[wid, sb], dst_v, sem_gb)
      ix0.wait()
      ix1.wait()

      @pl.loop(0, SCH, step=2 * KB_)
      def group(j):
        ga = [pltpu.async_copy(h_hbm.at[src_v.at[j + b]], rows[b], sem_ga)
              for b in range(KB_)]
        cs = []
        if with_counts:
          cs = [pltpu.async_copy(ones_v, cnt_sh.at[dst_v.at[j + b]], sem_c,
                                 add=True) for b in range(2 * KB_)]
        sa = []
        for b in range(KB_):
          ga[b].wait()
          sa.append(pltpu.async_copy(rows[b], acc_sh.at[dst_v.at[j + b]],
                                     sem_sa, add=True))
        gb = [pltpu.async_copy(h_hbm.at[src_v.at[j + KB_ + b]],
                               rows[KB_ + b], sem_gb)
              for b in range(KB_)]
        for d in sa:
          d.wait()
        sb_ = []
        for b in range(KB_):
          gb[b].wait()
          sb_.append(pltpu.async_copy(rows[KB_ + b],
                                      acc_sh.at[dst_v.at[j + KB_ + b]],
                                      sem_sb, add=True))
        for d in sb_:
          d.wait()
        for d in cs:
          d.wait()

    plsc.subcore_barrier()

    pltpu.sync_copy(acc_sh.at[pl.ds(base, RPT)],
                    acc_out.at[c, pl.ds(base, RPT)])
    if with_counts:
      pltpu.sync_copy(cnt_sh.at[pl.ds(base, RPT)],
                      cnt_out.at[c, pl.ds(base, RPT)])

  return pl.kernel(body, out_type=tuple(out_type), mesh=mesh,
                   scratch_types=tuple(scratch),
                   compiler_params=pltpu.CompilerParams(
                       use_tc_tiling_on_sc=False))


BN = 1280
NBLK = NP // BN


def _dense_body(h_ref, acc_ref, cnt_ref, wl_ref, bl_ref, wr_ref, out_ref):
  acc = acc_ref[0] + acc_ref[1]
  cnt = cnt_ref[0, :, 0:1] + cnt_ref[1, :, 0:1]
  aggr = acc * (1.0 / jnp.maximum(cnt, 1.0))
  z = (jnp.dot(aggr, wl_ref[...], preferred_element_type=_f32)
       + jnp.dot(h_ref[...], wr_ref[...], preferred_element_type=_f32)
       + bl_ref[...])
  out_ref[...] = jnp.maximum(z, 0.0)


_dense = pl.pallas_call(
    _dense_body,
    grid=(NBLK,),
    in_specs=[
        pl.BlockSpec((BN, F), lambda i: (i, 0)),
        pl.BlockSpec((NC, BN, F), lambda i: (0, i, 0)),
        pl.BlockSpec((NC, BN, 16), lambda i: (0, i, 0)),
        pl.BlockSpec((F, H), lambda i: (0, 0)),
        pl.BlockSpec((1, H), lambda i: (0, 0)),
        pl.BlockSpec((F, H), lambda i: (0, 0)),
    ],
    out_specs=pl.BlockSpec((BN, H), lambda i: (i, 0)),
    out_shape=jax.ShapeDtypeStruct((NP, H), _f32),
    compiler_params=pltpu.CompilerParams(
        dimension_semantics=("arbitrary",)),
)


def _pool_body(b_ref, h1_ref, h2_ref, h3_ref, w1_ref, b1_ref, w2_ref, b2_ref,
               out_ref, s_acc, c_acc):
  i = pl.program_id(0)

  @pl.when(i == 0)
  def _init():
    s_acc[...] = jnp.zeros_like(s_acc)
    c_acc[...] = jnp.zeros_like(c_acc)

  gids = lax.broadcasted_iota(jnp.int32, (G, 1), 0)
  oh = (b_ref[...] == gids).astype(_f32)
  s_acc[:, 0:H] += jnp.dot(oh, h1_ref[...], preferred_element_type=_f32)
  s_acc[:, H:2 * H] += jnp.dot(oh, h2_ref[...], preferred_element_type=_f32)
  s_acc[:, 2 * H:3 * H] += jnp.dot(oh, h3_ref[...], preferred_element_type=_f32)
  c_acc[...] += jnp.sum(oh, axis=1, keepdims=True)

  @pl.when(i == NBLK - 1)
  def _final():
    pooled = s_acc[...] / jnp.maximum(c_acc[...], 1.0)
    z = jnp.maximum(
        jnp.dot(pooled, w1_ref[...], preferred_element_type=_f32)
        + b1_ref[...], 0.0)
    z2 = jnp.dot(z, w2_ref[...], preferred_element_type=_f32) + b2_ref[...]
    m = jnp.max(z2, axis=1, keepdims=True)
    lse = jnp.log(jnp.sum(jnp.exp(z2 - m), axis=1, keepdims=True)) + m
    out_ref[...] = z2 - lse


_pool = pl.pallas_call(
    _pool_body,
    grid=(NBLK,),
    in_specs=[
        pl.BlockSpec((1, BN), lambda i: (0, i)),
        pl.BlockSpec((BN, H), lambda i: (i, 0)),
        pl.BlockSpec((BN, H), lambda i: (i, 0)),
        pl.BlockSpec((BN, H), lambda i: (i, 0)),
        pl.BlockSpec((3 * H, H), lambda i: (0, 0)),
        pl.BlockSpec((1, H), lambda i: (0, 0)),
        pl.BlockSpec((H, C), lambda i: (0, 0)),
        pl.BlockSpec((1, C), lambda i: (0, 0)),
    ],
    out_specs=pl.BlockSpec((G, C), lambda i: (0, 0)),
    out_shape=jax.ShapeDtypeStruct((G, C), _f32),
    scratch_shapes=[
        pltpu.VMEM((G, 3 * H), _f32),
        pltpu.VMEM((G, 1), _f32),
    ],
    compiler_params=pltpu.CompilerParams(
        dimension_semantics=("arbitrary",)),
)


@jax.jit
def kernel(x, edge_index, batch, Wl1, bl1, Wr1, Wl2, bl2, Wr2, Wl3, bl3, Wr3,
           W_lin1, b_lin1, W_lin2, b_lin2):
  src = edge_index[0].reshape(NW, NSB, SCH, CH)
  dst = edge_index[1].reshape(NW, NSB, SCH, CH)
  xp = jnp.pad(x, ((0, NP - N), (0, 0)))
  zrows = jnp.zeros((RPT, F), _f32)
  zcnt = jnp.zeros((RPT, 16), _f32)
  ones16 = jnp.ones((CH, 16), _f32)

  acc1, cnt = _make_sc_agg(True)(src, dst, xp, zrows, zcnt, ones16)
  h1 = _dense(xp, acc1, cnt, Wl1.T, bl1.reshape(1, H), Wr1.T)
  (acc2,) = _make_sc_agg(False)(src, dst, h1, zrows, zcnt, ones16)
  h2 = _dense(h1, acc2, cnt, Wl2.T, bl2.reshape(1, H), Wr2.T)
  (acc3,) = _make_sc_agg(False)(src, dst, h2, zrows, zcnt, ones16)
  h3 = _dense(h2, acc3, cnt, Wl3.T, bl3.reshape(1, H), Wr3.T)

  batch_p = jnp.pad(batch, (0, NP - N), constant_values=G).reshape(1, NP)
  return _pool(batch_p, h1, h2, h3, W_lin1.T, b_lin1.reshape(1, H),
               W_lin2.T, b_lin2.reshape(1, C))


# speedup vs baseline: 1.1333x; 1.1333x over previous
import functools

import jax
import jax.numpy as jnp
from jax import lax
from jax.experimental import pallas as pl
from jax.experimental.pallas import tpu as pltpu
from jax.experimental.pallas import tpu_sc as plsc

N = 10000
E = 320000
F = 128
H = 128
C = 32
G = 64

NC = 2
NS = 16
NW = NC * NS
EPW = E // NW
CH = 50
NCH = EPW // CH
SCH = 20
NSB = NCH // SCH
KB_ = 5
NP = 10240
RPT = NP // NS

_f32 = jnp.float32


@functools.cache
def _make_sc_agg(with_counts: bool):
  mesh = plsc.VectorSubcoreMesh(core_axis_name="c", subcore_axis_name="s")
  out_type = [jax.ShapeDtypeStruct((NC, NP, F), _f32)]
  scratch = [
      pltpu.VMEM((SCH, CH), jnp.int32),
      pltpu.VMEM((SCH, CH), jnp.int32),
  ] + [
      pltpu.VMEM((CH, F), _f32) for _ in range(KB_)
  ] + [
      pltpu.VMEM_SHARED((NP, F), _f32),
      pltpu.SemaphoreType.DMA,
      pltpu.SemaphoreType.DMA,
  ]
  if with_counts:
    out_type.append(jax.ShapeDtypeStruct((NC, NP, 16), _f32))
    scratch += [
        pltpu.VMEM((CH, 16), _f32),
        pltpu.VMEM((CH, 16), _f32),
        pltpu.VMEM_SHARED((NP, 16), _f32),
        pltpu.SemaphoreType.DMA,
    ]

  def body(src_hbm, dst_hbm, h_hbm, zrows_hbm, zcnt_hbm, ones_hbm, *refs):
    if with_counts:
      (acc_out, cnt_out, src_v, dst_v, *rest) = refs
      rows = rest[:KB_]
      acc_sh, sem_g, sem_s, ones_v, zc_v, cnt_sh, sem_c = rest[KB_:]
    else:
      (acc_out, src_v, dst_v, *rest) = refs
      rows = rest[:KB_]
      acc_sh, sem_g, sem_s = rest[KB_:]
    c = lax.axis_index("c")
    s = lax.axis_index("s")
    wid = c * NS + s
    base = s * RPT

    pltpu.sync_copy(zrows_hbm, acc_sh.at[pl.ds(base, RPT)])
    if with_counts:
      pltpu.sync_copy(zcnt_hbm, cnt_sh.at[pl.ds(base, RPT)])
      pltpu.sync_copy(ones_hbm, ones_v)
    plsc.subcore_barrier()

    @pl.loop(0, NSB)
    def superblock(sb):
      ix0 = pltpu.async_copy(src_hbm.at[wid, sb], src_v, sem_g)
      ix1 = pltpu.async_copy(dst_hbm.at[wid, sb], dst_v, sem_s)
      ix0.wait()
      ix1.wait()

      @pl.loop(0, SCH, step=KB_)
      def group(j):
        gs = [pltpu.async_copy(h_hbm.at[src_v.at[j + b]], rows[b], sem_g)
              for b in range(KB_)]
        cs = []
        if with_counts:
          cs = [pltpu.async_copy(ones_v, cnt_sh.at[dst_v.at[j + b]], sem_c,
                                 add=True) for b in range(KB_)]
        ss = []
        for b in range(KB_):
          gs[b].wait()
          ss.append(pltpu.async_copy(rows[b], acc_sh.at[dst_v.at[j + b]],
                                     sem_s, add=True))
        for d in ss:
          d.wait()
        for d in cs:
          d.wait()

    plsc.subcore_barrier()

    pltpu.sync_copy(acc_sh.at[pl.ds(base, RPT)],
                    acc_out.at[c, pl.ds(base, RPT)])
    if with_counts:
      pltpu.sync_copy(cnt_sh.at[pl.ds(base, RPT)],
                      cnt_out.at[c, pl.ds(base, RPT)])

  return pl.kernel(body, out_type=tuple(out_type), mesh=mesh,
                   scratch_types=tuple(scratch),
                   compiler_params=pltpu.CompilerParams(
                       use_tc_tiling_on_sc=False))


BN = 1280
NBLK = NP // BN


def _dense_body(h_ref, acc_ref, cnt_ref, wl_ref, bl_ref, wr_ref, out_ref):
  acc = acc_ref[0] + acc_ref[1]
  cnt = cnt_ref[0, :, 0:1] + cnt_ref[1, :, 0:1]
  aggr = acc * (1.0 / jnp.maximum(cnt, 1.0))
  z = (jnp.dot(aggr, wl_ref[...], preferred_element_type=_f32)
       + jnp.dot(h_ref[...], wr_ref[...], preferred_element_type=_f32)
       + bl_ref[...])
  out_ref[...] = jnp.maximum(z, 0.0)


_dense = pl.pallas_call(
    _dense_body,
    grid=(NBLK,),
    in_specs=[
        pl.BlockSpec((BN, F), lambda i: (i, 0)),
        pl.BlockSpec((NC, BN, F), lambda i: (0, i, 0)),
        pl.BlockSpec((NC, BN, 16), lambda i: (0, i, 0)),
        pl.BlockSpec((F, H), lambda i: (0, 0)),
        pl.BlockSpec((1, H), lambda i: (0, 0)),
        pl.BlockSpec((F, H), lambda i: (0, 0)),
    ],
    out_specs=pl.BlockSpec((BN, H), lambda i: (i, 0)),
    out_shape=jax.ShapeDtypeStruct((NP, H), _f32),
    compiler_params=pltpu.CompilerParams(
        dimension_semantics=("arbitrary",)),
)


def _pool_body(b_ref, h1_ref, h2_ref, h3_ref, w1_ref, b1_ref, w2_ref, b2_ref,
               out_ref, s_acc, c_acc):
  i = pl.program_id(0)

  @pl.when(i == 0)
  def _init():
    s_acc[...] = jnp.zeros_like(s_acc)
    c_acc[...] = jnp.zeros_like(c_acc)

  gids = lax.broadcasted_iota(jnp.int32, (G, 1), 0)
  oh = (b_ref[...] == gids).astype(_f32)
  s_acc[:, 0:H] += jnp.dot(oh, h1_ref[...], preferred_element_type=_f32)
  s_acc[:, H:2 * H] += jnp.dot(oh, h2_ref[...], preferred_element_type=_f32)
  s_acc[:, 2 * H:3 * H] += jnp.dot(oh, h3_ref[...], preferred_element_type=_f32)
  c_acc[...] += jnp.sum(oh, axis=1, keepdims=True)

  @pl.when(i == NBLK - 1)
  def _final():
    pooled = s_acc[...] / jnp.maximum(c_acc[...], 1.0)
    z = jnp.maximum(
        jnp.dot(pooled, w1_ref[...], preferred_element_type=_f32)
        + b1_ref[...], 0.0)
    z2 = jnp.dot(z, w2_ref[...], preferred_element_type=_f32) + b2_ref[...]
    m = jnp.max(z2, axis=1, keepdims=True)
    lse = jnp.log(jnp.sum(jnp.exp(z2 - m), axis=1, keepdims=True)) + m
    out_ref[...] = z2 - lse


_pool = pl.pallas_call(
    _pool_body,
    grid=(NBLK,),
    in_specs=[
        pl.BlockSpec((1, BN), lambda i: (0, i)),
        pl.BlockSpec((BN, H), lambda i: (i, 0)),
        pl.BlockSpec((BN, H), lambda i: (i, 0)),
        pl.BlockSpec((BN, H), lambda i: (i, 0)),
        pl.BlockSpec((3 * H, H), lambda i: (0, 0)),
        pl.BlockSpec((1, H), lambda i: (0, 0)),
        pl.BlockSpec((H, C), lambda i: (0, 0)),
        pl.BlockSpec((1, C), lambda i: (0, 0)),
    ],
    out_specs=pl.BlockSpec((G, C), lambda i: (0, 0)),
    out_shape=jax.ShapeDtypeStruct((G, C), _f32),
    scratch_shapes=[
        pltpu.VMEM((G, 3 * H), _f32),
        pltpu.VMEM((G, 1), _f32),
    ],
    compiler_params=pltpu.CompilerParams(
        dimension_semantics=("arbitrary",)),
)


@jax.jit
def kernel(x, edge_index, batch, Wl1, bl1, Wr1, Wl2, bl2, Wr2, Wl3, bl3, Wr3,
           W_lin1, b_lin1, W_lin2, b_lin2):
  src = edge_index[0].reshape(NW, NSB, SCH, CH)
  dst = edge_index[1].reshape(NW, NSB, SCH, CH)
  xp = jnp.pad(x, ((0, NP - N), (0, 0)))
  zrows = jnp.zeros((RPT, F), _f32)
  zcnt = jnp.zeros((RPT, 16), _f32)
  ones16 = jnp.ones((CH, 16), _f32)

  acc1, cnt = _make_sc_agg(True)(src, dst, xp, zrows, zcnt, ones16)
  h1 = _dense(xp, acc1, cnt, Wl1.T, bl1.reshape(1, H), Wr1.T)
  (acc2,) = _make_sc_agg(False)(src, dst, h1, zrows, zcnt, ones16)
  h2 = _dense(h1, acc2, cnt, Wl2.T, bl2.reshape(1, H), Wr2.T)
  (acc3,) = _make_sc_agg(False)(src, dst, h2, zrows, zcnt, ones16)
  h3 = _dense(h2, acc3, cnt, Wl3.T, bl3.reshape(1, H), Wr3.T)

  batch_p = jnp.pad(batch, (0, NP - N), constant_values=G).reshape(1, NP)
  return _pool(batch_p, h1, h2, h3, W_lin1.T, b_lin1.reshape(1, H),
               W_lin2.T, b_lin2.reshape(1, C))


# speedup vs baseline: 1.1471x; 1.0122x over previous
import functools

import jax
import jax.numpy as jnp
from jax import lax
from jax.experimental import pallas as pl
from jax.experimental.pallas import tpu as pltpu
from jax.experimental.pallas import tpu_sc as plsc

N = 10000
E = 320000
F = 128
H = 128
C = 32
G = 64

NC = 2
NS = 16
NW = NC * NS
EPW = E // NW
CH = 50
NCH = EPW // CH
SCH = 20
NSB = NCH // SCH
KB_ = 5
NP = 10240
RPT = NP // NS

_f32 = jnp.float32


@functools.cache
def _make_sc_agg(with_counts: bool):
  mesh = plsc.VectorSubcoreMesh(core_axis_name="c", subcore_axis_name="s")
  out_type = [jax.ShapeDtypeStruct((NC, NP, F), _f32)]
  scratch = [
      pltpu.VMEM((SCH, CH), jnp.int32),
      pltpu.VMEM((SCH, CH), jnp.int32),
      pltpu.VMEM((SCH, CH), jnp.int32),
      pltpu.VMEM((SCH, CH), jnp.int32),
      pltpu.SemaphoreType.DMA,
  ] + [
      pltpu.VMEM((CH, F), _f32) for _ in range(KB_)
  ] + [
      pltpu.VMEM_SHARED((NP, F), _f32),
      pltpu.SemaphoreType.DMA,
      pltpu.SemaphoreType.DMA,
  ]
  if with_counts:
    out_type.append(jax.ShapeDtypeStruct((NC, NP, 16), _f32))
    scratch += [
        pltpu.VMEM((CH, 16), _f32),
        pltpu.VMEM((CH, 16), _f32),
        pltpu.VMEM_SHARED((NP, 16), _f32),
        pltpu.SemaphoreType.DMA,
    ]

  def body(src_hbm, dst_hbm, h_hbm, zrows_hbm, zcnt_hbm, ones_hbm, *refs):
    if with_counts:
      (acc_out, cnt_out, src_a, dst_a, src_b, dst_b, sem_ix, *rest) = refs
      rows = rest[:KB_]
      acc_sh, sem_g, sem_s, ones_v, zc_v, cnt_sh, sem_c = rest[KB_:]
    else:
      (acc_out, src_a, dst_a, src_b, dst_b, sem_ix, *rest) = refs
      rows = rest[:KB_]
      acc_sh, sem_g, sem_s = rest[KB_:]
    idx_bufs = [(src_a, dst_a), (src_b, dst_b)]
    c = lax.axis_index("c")
    s = lax.axis_index("s")
    wid = c * NS + s
    base = s * RPT

    pltpu.sync_copy(zrows_hbm, acc_sh.at[pl.ds(base, RPT)])
    if with_counts:
      pltpu.sync_copy(zcnt_hbm, cnt_sh.at[pl.ds(base, RPT)])
      pltpu.sync_copy(ones_hbm, ones_v)
    plsc.subcore_barrier()

    pltpu.sync_copy(src_hbm.at[wid, 0], src_a)
    pltpu.sync_copy(dst_hbm.at[wid, 0], dst_a)

    for sb in range(NSB):
      src_v, dst_v = idx_bufs[sb % 2]
      nsrc_v, ndst_v = idx_bufs[(sb + 1) % 2]
      pf = []
      if sb + 1 < NSB:
        pf = [pltpu.async_copy(src_hbm.at[wid, sb + 1], nsrc_v, sem_ix),
              pltpu.async_copy(dst_hbm.at[wid, sb + 1], ndst_v, sem_ix)]

      @pl.loop(0, SCH, step=KB_)
      def group(j):
        gs = [pltpu.async_copy(h_hbm.at[src_v.at[j + b]], rows[b], sem_g)
              for b in range(KB_)]
        cs = []
        if with_counts:
          cs = [pltpu.async_copy(ones_v, cnt_sh.at[dst_v.at[j + b]], sem_c,
                                 add=True) for b in range(KB_)]
        ss = []
        for b in range(KB_):
          gs[b].wait()
          ss.append(pltpu.async_copy(rows[b], acc_sh.at[dst_v.at[j + b]],
                                     sem_s, add=True))
        for d in ss:
          d.wait()
        for d in cs:
          d.wait()

      for d in pf:
        d.wait()

    plsc.subcore_barrier()

    pltpu.sync_copy(acc_sh.at[pl.ds(base, RPT)],
                    acc_out.at[c, pl.ds(base, RPT)])
    if with_counts:
      pltpu.sync_copy(cnt_sh.at[pl.ds(base, RPT)],
                      cnt_out.at[c, pl.ds(base, RPT)])

  return pl.kernel(body, out_type=tuple(out_type), mesh=mesh,
                   scratch_types=tuple(scratch),
                   compiler_params=pltpu.CompilerParams(
                       use_tc_tiling_on_sc=False))


BN = 1280
NBLK = NP // BN


def _dense_body(h_ref, acc_ref, cnt_ref, wl_ref, bl_ref, wr_ref, out_ref):
  acc = acc_ref[0] + acc_ref[1]
  cnt = cnt_ref[0, :, 0:1] + cnt_ref[1, :, 0:1]
  aggr = acc * (1.0 / jnp.maximum(cnt, 1.0))
  z = (jnp.dot(aggr, wl_ref[...], preferred_element_type=_f32)
       + jnp.dot(h_ref[...], wr_ref[...], preferred_element_type=_f32)
       + bl_ref[...])
  out_ref[...] = jnp.maximum(z, 0.0)


_dense = pl.pallas_call(
    _dense_body,
    grid=(NBLK,),
    in_specs=[
        pl.BlockSpec((BN, F), lambda i: (i, 0)),
        pl.BlockSpec((NC, BN, F), lambda i: (0, i, 0)),
        pl.BlockSpec((NC, BN, 16), lambda i: (0, i, 0)),
        pl.BlockSpec((F, H), lambda i: (0, 0)),
        pl.BlockSpec((1, H), lambda i: (0, 0)),
        pl.BlockSpec((F, H), lambda i: (0, 0)),
    ],
    out_specs=pl.BlockSpec((BN, H), lambda i: (i, 0)),
    out_shape=jax.ShapeDtypeStruct((NP, H), _f32),
    compiler_params=pltpu.CompilerParams(
        dimension_semantics=("arbitrary",)),
)


def _pool_body(b_ref, h1_ref, h2_ref, h3_ref, w1_ref, b1_ref, w2_ref, b2_ref,
               out_ref, s_acc, c_acc):
  i = pl.program_id(0)

  @pl.when(i == 0)
  def _init():
    s_acc[...] = jnp.zeros_like(s_acc)
    c_acc[...] = jnp.zeros_like(c_acc)

  gids = lax.broadcasted_iota(jnp.int32, (G, 1), 0)
  oh = (b_ref[...] == gids).astype(_f32)
  s_acc[:, 0:H] += jnp.dot(oh, h1_ref[...], preferred_element_type=_f32)
  s_acc[:, H:2 * H] += jnp.dot(oh, h2_ref[...], preferred_element_type=_f32)
  s_acc[:, 2 * H:3 * H] += jnp.dot(oh, h3_ref[...], preferred_element_type=_f32)
  c_acc[...] += jnp.sum(oh, axis=1, keepdims=True)

  @pl.when(i == NBLK - 1)
  def _final():
    pooled = s_acc[...] / jnp.maximum(c_acc[...], 1.0)
    z = jnp.maximum(
        jnp.dot(pooled, w1_ref[...], preferred_element_type=_f32)
        + b1_ref[...], 0.0)
    z2 = jnp.dot(z, w2_ref[...], preferred_element_type=_f32) + b2_ref[...]
    m = jnp.max(z2, axis=1, keepdims=True)
    lse = jnp.log(jnp.sum(jnp.exp(z2 - m), axis=1, keepdims=True)) + m
    out_ref[...] = z2 - lse


_pool = pl.pallas_call(
    _pool_body,
    grid=(NBLK,),
    in_specs=[
        pl.BlockSpec((1, BN), lambda i: (0, i)),
        pl.BlockSpec((BN, H), lambda i: (i, 0)),
        pl.BlockSpec((BN, H), lambda i: (i, 0)),
        pl.BlockSpec((BN, H), lambda i: (i, 0)),
        pl.BlockSpec((3 * H, H), lambda i: (0, 0)),
        pl.BlockSpec((1, H), lambda i: (0, 0)),
        pl.BlockSpec((H, C), lambda i: (0, 0)),
        pl.BlockSpec((1, C), lambda i: (0, 0)),
    ],
    out_specs=pl.BlockSpec((G, C), lambda i: (0, 0)),
    out_shape=jax.ShapeDtypeStruct((G, C), _f32),
    scratch_shapes=[
        pltpu.VMEM((G, 3 * H), _f32),
        pltpu.VMEM((G, 1), _f32),
    ],
    compiler_params=pltpu.CompilerParams(
        dimension_semantics=("arbitrary",)),
)


@jax.jit
def kernel(x, edge_index, batch, Wl1, bl1, Wr1, Wl2, bl2, Wr2, Wl3, bl3, Wr3,
           W_lin1, b_lin1, W_lin2, b_lin2):
  src = edge_index[0].reshape(NW, NSB, SCH, CH)
  dst = edge_index[1].reshape(NW, NSB, SCH, CH)
  xp = jnp.pad(x, ((0, NP - N), (0, 0)))
  zrows = jnp.zeros((RPT, F), _f32)
  zcnt = jnp.zeros((RPT, 16), _f32)
  ones16 = jnp.ones((CH, 16), _f32)

  acc1, cnt = _make_sc_agg(True)(src, dst, xp, zrows, zcnt, ones16)
  h1 = _dense(xp, acc1, cnt, Wl1.T, bl1.reshape(1, H), Wr1.T)
  (acc2,) = _make_sc_agg(False)(src, dst, h1, zrows, zcnt, ones16)
  h2 = _dense(h1, acc2, cnt, Wl2.T, bl2.reshape(1, H), Wr2.T)
  (acc3,) = _make_sc_agg(False)(src, dst, h2, zrows, zcnt, ones16)
  h3 = _dense(h2, acc3, cnt, Wl3.T, bl3.reshape(1, H), Wr3.T)

  batch_p = jnp.pad(batch, (0, NP - N), constant_values=G).reshape(1, NP)
  return _pool(batch_p, h1, h2, h3, W_lin1.T, b_lin1.reshape(1, H),
               W_lin2.T, b_lin2.reshape(1, C))
